# baseline (device time: 10382 ns/iter reference)
import jax
import jax.numpy as jnp
from jax import lax
from jax.experimental import pallas as pl
from jax.experimental.pallas import tpu as pltpu

N_DEV = 4
C_GLOBAL = 512.0
EPS = 1e-5


def kernel(x, t_emb, W_scale, W_shift):
    b, s, c = x.shape

    def body(x_ref, t_ref, wsc_ref, wsh_ref, out_ref,
             my_stats, comm_ref, send_sems, recv_sems):
        my = lax.axis_index("i")

        barrier_sem = pltpu.get_barrier_semaphore()
        for d in range(1, N_DEV):
            pl.semaphore_signal(
                barrier_sem, inc=1,
                device_id=((my + d) % N_DEV,),
                device_id_type=pl.DeviceIdType.MESH,
            )
        pl.semaphore_wait(barrier_sem, N_DEV - 1)

        xv = x_ref[...]
        my_stats[0] = jnp.sum(xv, axis=2)
        my_stats[1] = jnp.sum(xv * xv, axis=2)

        rdmas = []
        for d in range(1, N_DEV):
            rdma = pltpu.make_async_remote_copy(
                src_ref=my_stats,
                dst_ref=comm_ref.at[d - 1],
                send_sem=send_sems.at[d - 1],
                recv_sem=recv_sems.at[d - 1],
                device_id=((my + d) % N_DEV,),
                device_id_type=pl.DeviceIdType.MESH,
            )
            rdma.start()
            rdmas.append(rdma)

        tv = t_ref[...]
        scale = jnp.dot(tv, wsc_ref[...], preferred_element_type=jnp.float32)
        shift = jnp.dot(tv, wsh_ref[...], preferred_element_type=jnp.float32)

        for rdma in rdmas:
            rdma.wait()

        tot = my_stats[...] + comm_ref[0] + comm_ref[1] + comm_ref[2]
        mean = tot[0] * (1.0 / C_GLOBAL)
        var = tot[1] * (1.0 / C_GLOBAL) - mean * mean
        rstd = lax.rsqrt(var + EPS)
        h = (xv - mean[:, :, None]) * rstd[:, :, None]
        out_ref[...] = h * (1.0 + scale[:, None, :]) + shift[:, None, :]

    return pl.pallas_call(
        body,
        out_shape=jax.ShapeDtypeStruct((b, s, c), jnp.float32),
        in_specs=[pl.BlockSpec(memory_space=pltpu.VMEM)] * 4,
        out_specs=pl.BlockSpec(memory_space=pltpu.VMEM),
        scratch_shapes=[
            pltpu.VMEM((2, b, s), jnp.float32),
            pltpu.VMEM((N_DEV - 1, 2, b, s), jnp.float32),
            pltpu.SemaphoreType.DMA((N_DEV - 1,)),
            pltpu.SemaphoreType.DMA((N_DEV - 1,)),
        ],
        compiler_params=pltpu.CompilerParams(collective_id=0),
    )(x, t_emb, W_scale, W_shift)
